# baseline (speedup 1.0000x reference)
import jax
import jax.numpy as jnp
from jax import lax
from jax.experimental import pallas as pl
from jax.experimental.pallas import tpu as pltpu


def _body(tok_ref, intent_ref, emb_hbm,
          dW_ref, db_ref, sW_ref, sb_ref, ddW_ref, ddb_ref, sdW_ref, sdb_ref,
          intent_out, slots_out, gbuf, sem):
    S = gbuf.shape[0]
    B_half = pl.num_programs(1)
    b = pl.program_id(0) * B_half + pl.program_id(1)

    for w in range(S):
        pltpu.make_async_copy(emb_hbm.at[tok_ref[b, w]], gbuf.at[w], sem).start()
    pltpu.make_async_copy(emb_hbm.at[pl.ds(0, S)], gbuf, sem).wait()

    e0 = gbuf[:, 0, :]
    e1 = gbuf[:, 1, :]

    we = jnp.maximum(
        jnp.dot(e0, sW_ref[0:128, :], preferred_element_type=jnp.float32)
        + jnp.dot(e1, sW_ref[128:256, :], preferred_element_type=jnp.float32)
        + sb_ref[...], 0.0)

    iid = intent_ref[b]
    onehot = (lax.broadcasted_iota(jnp.int32, (1, sdW_ref.shape[0] - 256), 1)
              == iid).astype(jnp.float32)
    bot = jnp.dot(onehot, sdW_ref[256:, :], preferred_element_type=jnp.float32)
    slots_out[...] = (
        jnp.dot(we, sdW_ref[0:256, :], preferred_element_type=jnp.float32)
        + bot + sdb_ref[...])

    s0 = jnp.mean(e0, axis=0, keepdims=True)
    s1 = jnp.mean(e1, axis=0, keepdims=True)
    se = jnp.maximum(
        jnp.dot(s0, dW_ref[0:128, :], preferred_element_type=jnp.float32)
        + jnp.dot(s1, dW_ref[128:256, :], preferred_element_type=jnp.float32)
        + db_ref[...], 0.0)
    logits = (jnp.dot(se, ddW_ref[...], preferred_element_type=jnp.float32)
              + ddb_ref[...])
    intent_out[...] = logits.reshape(intent_out.shape)


def kernel(token_ids, all_intents, emb_table, doc_enc_W, doc_enc_b,
           slot_enc_W, slot_enc_b, doc_dec_W, doc_dec_b,
           slot_dec_W, slot_dec_b):
    B, S = token_ids.shape
    VOCAB, EMB = emb_table.shape
    ENC = doc_enc_W.shape[1]
    N_INTENTS = doc_dec_W.shape[1]
    N_SLOTS = slot_dec_W.shape[1]

    tok = token_ids.astype(jnp.int32)
    intents = all_intents.astype(jnp.int32)
    emb3 = emb_table.reshape(VOCAB, EMB // 128, 128)

    half = B // 2
    grid = (2, half)

    def _fixed(c, j, *_):
        return (0, 0)

    def _row(c, j, *_):
        return (c * half + j, 0)

    def _row3(c, j, *_):
        return (c * half + j, 0, 0)

    grid_spec = pltpu.PrefetchScalarGridSpec(
        num_scalar_prefetch=2,
        grid=grid,
        in_specs=[
            pl.BlockSpec(memory_space=pl.ANY),
            pl.BlockSpec((EMB, ENC), _fixed),
            pl.BlockSpec((1, ENC), _fixed),
            pl.BlockSpec((EMB, ENC), _fixed),
            pl.BlockSpec((1, ENC), _fixed),
            pl.BlockSpec((ENC, N_INTENTS), _fixed),
            pl.BlockSpec((1, N_INTENTS), _fixed),
            pl.BlockSpec((slot_dec_W.shape[0], N_SLOTS), _fixed),
            pl.BlockSpec((1, N_SLOTS), _fixed),
        ],
        out_specs=[
            pl.BlockSpec((1, 1, N_INTENTS), _row3),
            pl.BlockSpec((S, N_SLOTS), _row),
        ],
        scratch_shapes=[
            pltpu.VMEM((S, EMB // 128, 128), jnp.float32),
            pltpu.SemaphoreType.DMA,
        ],
    )

    intent_batch, slots_batch = pl.pallas_call(
        _body,
        out_shape=[
            jax.ShapeDtypeStruct((B, 1, N_INTENTS), jnp.float32),
            jax.ShapeDtypeStruct((B * S, N_SLOTS), jnp.float32),
        ],
        grid_spec=grid_spec,
        compiler_params=pltpu.CompilerParams(
            dimension_semantics=("parallel", "arbitrary"),
        ),
        name="intent_slot_fused",
    )(tok, intents, emb3, doc_enc_W, doc_enc_b.reshape(1, ENC),
      slot_enc_W, slot_enc_b.reshape(1, ENC), doc_dec_W,
      doc_dec_b.reshape(1, N_INTENTS), slot_dec_W,
      slot_dec_b.reshape(1, N_SLOTS))
    return intent_batch.reshape(B, N_INTENTS), slots_batch


# speedup vs baseline: 1.4833x; 1.4833x over previous
import jax
import jax.numpy as jnp
from jax import lax
from jax.experimental import pallas as pl
from jax.experimental.pallas import tpu as pltpu

_CHUNK = 4


def _body(tok_ref, intent_ref, emb_hbm,
          dW_ref, db_ref, sW_ref, sb_ref, ddW_ref, ddb_ref, sdW_ref, sdb_ref,
          intent_out, slots_out, gbuf, sems):
    nj = pl.num_programs(1)
    C = _CHUNK
    S = tok_ref.shape[1]
    CS = C * S
    c = pl.program_id(0)
    j = pl.program_id(1)
    half = nj * C
    b0 = c * half

    def issue(chunk_idx, slot):
        base = slot * CS
        first = b0 + chunk_idx * C
        for wi in range(CS):
            tok = tok_ref[first + wi // S, wi % S]
            pltpu.make_async_copy(emb_hbm.at[tok], gbuf.at[base + wi],
                                  sems.at[slot]).start()

    @pl.when(j == 0)
    def _prologue():
        issue(j, 0)

    @pl.when(j + 1 < nj)
    def _prefetch():
        issue(j + 1, (j + 1) % 2)

    slot = lax.rem(j, 2)
    base = slot * CS
    pltpu.make_async_copy(emb_hbm.at[pl.ds(0, CS)],
                          gbuf.at[pl.ds(0, CS)], sems.at[slot]).wait()

    x3 = gbuf[pl.ds(base, CS)]
    e0 = x3[:, 0, :]
    e1 = x3[:, 1, :]

    we = jnp.maximum(
        jnp.dot(e0, sW_ref[0:128, :], preferred_element_type=jnp.float32)
        + jnp.dot(e1, sW_ref[128:256, :], preferred_element_type=jnp.float32)
        + sb_ref[...], 0.0)

    top = jnp.dot(we, sdW_ref[0:256, :], preferred_element_type=jnp.float32)

    n_int = sdW_ref.shape[0] - 256
    iota = lax.broadcasted_iota(jnp.int32, (1, n_int), 1)
    oh = jnp.concatenate(
        [(iota == intent_ref[b0 + j * C + r]).astype(jnp.float32)
         for r in range(C)], axis=0)
    bots = jnp.dot(oh, sdW_ref[256:, :], preferred_element_type=jnp.float32)

    n_slots = top.shape[1]
    top3 = top.reshape(C, S, n_slots)
    slots_out[...] = (top3 + bots[:, None, :] + sdb_ref[...]).reshape(CS, n_slots)

    m0 = jnp.mean(e0.reshape(C, S, 128), axis=1)
    m1 = jnp.mean(e1.reshape(C, S, 128), axis=1)
    se = jnp.maximum(
        jnp.dot(m0, dW_ref[0:128, :], preferred_element_type=jnp.float32)
        + jnp.dot(m1, dW_ref[128:256, :], preferred_element_type=jnp.float32)
        + db_ref[...], 0.0)
    logits = (jnp.dot(se, ddW_ref[...], preferred_element_type=jnp.float32)
              + ddb_ref[...])
    intent_out[...] = logits.reshape(intent_out.shape)


def kernel(token_ids, all_intents, emb_table, doc_enc_W, doc_enc_b,
           slot_enc_W, slot_enc_b, doc_dec_W, doc_dec_b,
           slot_dec_W, slot_dec_b):
    B, S = token_ids.shape
    VOCAB, EMB = emb_table.shape
    ENC = doc_enc_W.shape[1]
    N_INTENTS = doc_dec_W.shape[1]
    N_SLOTS = slot_dec_W.shape[1]
    C = _CHUNK

    tok = token_ids.astype(jnp.int32)
    intents = all_intents.astype(jnp.int32)
    emb3 = emb_table.reshape(VOCAB, EMB // 128, 128)

    half = B // 2
    nj = half // C
    grid = (2, nj)

    def _fixed(c, j, *_):
        return (0, 0)

    def _slots_map(c, j, *_):
        return (c * nj + j, 0)

    def _intent_map(c, j, *_):
        return (c * nj + j, 0, 0)

    grid_spec = pltpu.PrefetchScalarGridSpec(
        num_scalar_prefetch=2,
        grid=grid,
        in_specs=[
            pl.BlockSpec(memory_space=pl.ANY),
            pl.BlockSpec((EMB, ENC), _fixed),
            pl.BlockSpec((1, ENC), _fixed),
            pl.BlockSpec((EMB, ENC), _fixed),
            pl.BlockSpec((1, ENC), _fixed),
            pl.BlockSpec((ENC, N_INTENTS), _fixed),
            pl.BlockSpec((1, N_INTENTS), _fixed),
            pl.BlockSpec((slot_dec_W.shape[0], N_SLOTS), _fixed),
            pl.BlockSpec((1, N_SLOTS), _fixed),
        ],
        out_specs=[
            pl.BlockSpec((C, 1, N_INTENTS), _intent_map),
            pl.BlockSpec((C * S, N_SLOTS), _slots_map),
        ],
        scratch_shapes=[
            pltpu.VMEM((2 * C * S, EMB // 128, 128), jnp.float32),
            pltpu.SemaphoreType.DMA((2,)),
        ],
    )

    intent_batch, slots_batch = pl.pallas_call(
        _body,
        out_shape=[
            jax.ShapeDtypeStruct((B, 1, N_INTENTS), jnp.float32),
            jax.ShapeDtypeStruct((B * S, N_SLOTS), jnp.float32),
        ],
        grid_spec=grid_spec,
        compiler_params=pltpu.CompilerParams(
            dimension_semantics=("parallel", "arbitrary"),
        ),
        name="intent_slot_fused",
    )(tok, intents, emb3, doc_enc_W, doc_enc_b.reshape(1, ENC),
      slot_enc_W, slot_enc_b.reshape(1, ENC), doc_dec_W,
      doc_dec_b.reshape(1, N_INTENTS), slot_dec_W,
      slot_dec_b.reshape(1, N_SLOTS))
    return intent_batch.reshape(B, N_INTENTS), slots_batch


# speedup vs baseline: 3.5779x; 2.4121x over previous
import jax
import jax.numpy as jnp
from jax import lax
from jax.experimental import pallas as pl
from jax.experimental.pallas import tpu as pltpu

_CHUNK = 4


def _body(tok_ref, intent_ref, emb_hbm,
          dW_ref, db_ref, sW_ref, sb_ref, ddW_ref, ddb_ref, sdW_ref, sdb_ref,
          intent_out, slots_out, gbuf, xbuf, sems):
    nj = pl.num_programs(1)
    C = _CHUNK
    S = tok_ref.shape[1]
    CS = C * S
    c = pl.program_id(0)
    j = pl.program_id(1)
    half = nj * C
    b0 = c * half

    def issue(chunk_idx, slot):
        sbase = pl.multiple_of(slot * (CS * 8), 8)
        first = b0 + chunk_idx * C
        for wi in range(CS):
            tok = tok_ref[first + wi // S, wi % S]
            src = pl.multiple_of((tok >> 3) << 3, 8)
            pltpu.make_async_copy(
                emb_hbm.at[pl.ds(src, 8), :],
                gbuf.at[pl.ds(sbase + 8 * wi, 8), :],
                sems.at[slot]).start()

    @pl.when(j == 0)
    def _prologue():
        issue(j, 0)

    @pl.when(j + 1 < nj)
    def _prefetch():
        issue(j + 1, (j + 1) % 2)

    slot = lax.rem(j, 2)
    sbase = pl.multiple_of(slot * (CS * 8), 8)
    pltpu.make_async_copy(emb_hbm.at[pl.ds(0, CS * 8), :],
                          gbuf.at[pl.ds(0, CS * 8), :], sems.at[slot]).wait()

    first = b0 + j * C
    for wi in range(CS):
        tok = tok_ref[first + wi // S, wi % S]
        slab = gbuf[pl.ds(sbase + 8 * wi, 8), :]
        xbuf[pl.ds(wi, 1), :] = pltpu.roll(slab, -(tok & 7), axis=0)[0:1, :]

    x = xbuf[...]

    we = jnp.maximum(
        jnp.dot(x, sW_ref[...], preferred_element_type=jnp.float32)
        + sb_ref[...], 0.0)

    top = jnp.dot(we, sdW_ref[0:256, :], preferred_element_type=jnp.float32)

    n_int = sdW_ref.shape[0] - 256
    iota = lax.broadcasted_iota(jnp.int32, (1, n_int), 1)
    oh = jnp.concatenate(
        [(iota == intent_ref[first + r]).astype(jnp.float32)
         for r in range(C)], axis=0)
    bots = jnp.dot(oh, sdW_ref[256:, :], preferred_element_type=jnp.float32)

    n_slots = top.shape[1]
    top3 = top.reshape(C, S, n_slots)
    slots_out[...] = (top3 + bots[:, None, :] + sdb_ref[...]).reshape(CS, n_slots)

    m = jnp.mean(x.reshape(C, S, x.shape[1]), axis=1)
    se = jnp.maximum(
        jnp.dot(m, dW_ref[...], preferred_element_type=jnp.float32)
        + db_ref[...], 0.0)
    logits = (jnp.dot(se, ddW_ref[...], preferred_element_type=jnp.float32)
              + ddb_ref[...])
    intent_out[...] = logits.reshape(intent_out.shape)


def kernel(token_ids, all_intents, emb_table, doc_enc_W, doc_enc_b,
           slot_enc_W, slot_enc_b, doc_dec_W, doc_dec_b,
           slot_dec_W, slot_dec_b):
    B, S = token_ids.shape
    VOCAB, EMB = emb_table.shape
    ENC = doc_enc_W.shape[1]
    N_INTENTS = doc_dec_W.shape[1]
    N_SLOTS = slot_dec_W.shape[1]
    C = _CHUNK

    tok = token_ids.astype(jnp.int32)
    intents = all_intents.astype(jnp.int32)

    half = B // 2
    nj = half // C
    grid = (2, nj)

    def _fixed(c, j, *_):
        return (0, 0)

    def _slots_map(c, j, *_):
        return (c * nj + j, 0)

    def _intent_map(c, j, *_):
        return (c * nj + j, 0, 0)

    grid_spec = pltpu.PrefetchScalarGridSpec(
        num_scalar_prefetch=2,
        grid=grid,
        in_specs=[
            pl.BlockSpec(memory_space=pl.ANY),
            pl.BlockSpec((EMB, ENC), _fixed),
            pl.BlockSpec((1, ENC), _fixed),
            pl.BlockSpec((EMB, ENC), _fixed),
            pl.BlockSpec((1, ENC), _fixed),
            pl.BlockSpec((ENC, N_INTENTS), _fixed),
            pl.BlockSpec((1, N_INTENTS), _fixed),
            pl.BlockSpec((slot_dec_W.shape[0], N_SLOTS), _fixed),
            pl.BlockSpec((1, N_SLOTS), _fixed),
        ],
        out_specs=[
            pl.BlockSpec((C, 1, N_INTENTS), _intent_map),
            pl.BlockSpec((C * S, N_SLOTS), _slots_map),
        ],
        scratch_shapes=[
            pltpu.VMEM((2 * C * S * 8, EMB), jnp.float32),
            pltpu.VMEM((C * S, EMB), jnp.float32),
            pltpu.SemaphoreType.DMA((2,)),
        ],
    )

    intent_batch, slots_batch = pl.pallas_call(
        _body,
        out_shape=[
            jax.ShapeDtypeStruct((B, 1, N_INTENTS), jnp.float32),
            jax.ShapeDtypeStruct((B * S, N_SLOTS), jnp.float32),
        ],
        grid_spec=grid_spec,
        compiler_params=pltpu.CompilerParams(
            dimension_semantics=("parallel", "arbitrary"),
        ),
        name="intent_slot_fused",
    )(tok, intents, emb_table, doc_enc_W, doc_enc_b.reshape(1, ENC),
      slot_enc_W, slot_enc_b.reshape(1, ENC), doc_dec_W,
      doc_dec_b.reshape(1, N_INTENTS), slot_dec_W,
      slot_dec_b.reshape(1, N_SLOTS))
    return intent_batch.reshape(B, N_INTENTS), slots_batch


# speedup vs baseline: 4.1973x; 1.1731x over previous
import jax
import jax.numpy as jnp
from jax import lax
from jax.experimental import pallas as pl
from jax.experimental.pallas import tpu as pltpu

_CHUNK = 4


def _body(tok_ref, intent_ref, emb_hbm,
          dW_ref, db_ref, sW_ref, sb_ref, ddW_ref, ddb_ref, sdW_ref, sdb_ref,
          intent_out, slots_out, xbuf, sems):
    nj = pl.num_programs(1)
    C = _CHUNK
    S = tok_ref.shape[1]
    CS = C * S
    c = pl.program_id(0)
    j = pl.program_id(1)
    half = nj * C
    b0 = c * half

    def issue(chunk_idx, slot):
        sbase = pl.multiple_of(slot * CS, 8)
        first = b0 + chunk_idx * C
        for wi in range(CS):
            tok = tok_ref[first + wi // S, wi % S]
            pltpu.make_async_copy(
                emb_hbm.at[pl.ds(tok, 1), :],
                xbuf.at[pl.ds(sbase + wi, 1), :],
                sems.at[slot]).start()

    @pl.when(j == 0)
    def _prologue():
        issue(j, 0)

    @pl.when(j + 1 < nj)
    def _prefetch():
        issue(j + 1, (j + 1) % 2)

    slot = lax.rem(j, 2)
    base = pl.multiple_of(slot * CS, 8)
    pltpu.make_async_copy(emb_hbm.at[pl.ds(0, CS), :],
                          xbuf.at[pl.ds(0, CS), :], sems.at[slot]).wait()

    x = xbuf[pl.ds(base, CS), :]

    we = jnp.maximum(
        jnp.dot(x, sW_ref[...], preferred_element_type=jnp.float32)
        + sb_ref[...], 0.0)

    top = jnp.dot(we, sdW_ref[0:256, :], preferred_element_type=jnp.float32)

    first = b0 + j * C
    n_int = sdW_ref.shape[0] - 256
    iota = lax.broadcasted_iota(jnp.int32, (1, n_int), 1)
    oh = jnp.concatenate(
        [(iota == intent_ref[first + r]).astype(jnp.float32)
         for r in range(C)], axis=0)
    bots = jnp.dot(oh, sdW_ref[256:, :], preferred_element_type=jnp.float32)

    n_slots = top.shape[1]
    top3 = top.reshape(C, S, n_slots)
    slots_out[...] = (top3 + bots[:, None, :] + sdb_ref[...]).reshape(CS, n_slots)

    m = jnp.mean(x.reshape(C, S, x.shape[1]), axis=1)
    se = jnp.maximum(
        jnp.dot(m, dW_ref[...], preferred_element_type=jnp.float32)
        + db_ref[...], 0.0)
    logits = (jnp.dot(se, ddW_ref[...], preferred_element_type=jnp.float32)
              + ddb_ref[...])
    intent_out[...] = logits.reshape(intent_out.shape)


def kernel(token_ids, all_intents, emb_table, doc_enc_W, doc_enc_b,
           slot_enc_W, slot_enc_b, doc_dec_W, doc_dec_b,
           slot_dec_W, slot_dec_b):
    B, S = token_ids.shape
    VOCAB, EMB = emb_table.shape
    ENC = doc_enc_W.shape[1]
    N_INTENTS = doc_dec_W.shape[1]
    N_SLOTS = slot_dec_W.shape[1]
    C = _CHUNK

    tok = token_ids.astype(jnp.int32)
    intents = all_intents.astype(jnp.int32)

    half = B // 2
    nj = half // C
    grid = (2, nj)

    def _fixed(c, j, *_):
        return (0, 0)

    def _slots_map(c, j, *_):
        return (c * nj + j, 0)

    def _intent_map(c, j, *_):
        return (c * nj + j, 0, 0)

    grid_spec = pltpu.PrefetchScalarGridSpec(
        num_scalar_prefetch=2,
        grid=grid,
        in_specs=[
            pl.BlockSpec(memory_space=pl.ANY),
            pl.BlockSpec((EMB, ENC), _fixed),
            pl.BlockSpec((1, ENC), _fixed),
            pl.BlockSpec((EMB, ENC), _fixed),
            pl.BlockSpec((1, ENC), _fixed),
            pl.BlockSpec((ENC, N_INTENTS), _fixed),
            pl.BlockSpec((1, N_INTENTS), _fixed),
            pl.BlockSpec((slot_dec_W.shape[0], N_SLOTS), _fixed),
            pl.BlockSpec((1, N_SLOTS), _fixed),
        ],
        out_specs=[
            pl.BlockSpec((C, 1, N_INTENTS), _intent_map),
            pl.BlockSpec((C * S, N_SLOTS), _slots_map),
        ],
        scratch_shapes=[
            pltpu.VMEM((2 * C * S, EMB), jnp.float32),
            pltpu.SemaphoreType.DMA((2,)),
        ],
    )

    intent_batch, slots_batch = pl.pallas_call(
        _body,
        out_shape=[
            jax.ShapeDtypeStruct((B, 1, N_INTENTS), jnp.float32),
            jax.ShapeDtypeStruct((B * S, N_SLOTS), jnp.float32),
        ],
        grid_spec=grid_spec,
        compiler_params=pltpu.CompilerParams(
            dimension_semantics=("parallel", "arbitrary"),
        ),
        name="intent_slot_fused",
    )(tok, intents, emb_table, doc_enc_W, doc_enc_b.reshape(1, ENC),
      slot_enc_W, slot_enc_b.reshape(1, ENC), doc_dec_W,
      doc_dec_b.reshape(1, N_INTENTS), slot_dec_W,
      slot_dec_b.reshape(1, N_SLOTS))
    return intent_batch.reshape(B, N_INTENTS), slots_batch


# speedup vs baseline: 4.6256x; 1.1020x over previous
import jax
import jax.numpy as jnp
from jax import lax
from jax.experimental import pallas as pl
from jax.experimental.pallas import tpu as pltpu

_CHUNK = 8


def _body(tok_ref, intent_ref, emb_hbm,
          dW_ref, db_ref, sW_ref, sb_ref, ddW_ref, ddb_ref, sdW_ref, sdb_ref,
          intent_out, slots_out, xbuf, sems):
    nj = pl.num_programs(1)
    C = _CHUNK
    S = tok_ref.shape[1]
    CS = C * S
    c = pl.program_id(0)
    j = pl.program_id(1)
    half = nj * C
    b0 = c * half

    def issue(chunk_idx, slot):
        sbase = pl.multiple_of(slot * CS, 8)
        first = b0 + chunk_idx * C
        for wi in range(CS):
            tok = tok_ref[first + wi // S, wi % S]
            pltpu.make_async_copy(
                emb_hbm.at[pl.ds(tok, 1), :],
                xbuf.at[pl.ds(sbase + wi, 1), :],
                sems.at[slot]).start()

    @pl.when(j == 0)
    def _prologue():
        issue(j, 0)

    @pl.when(j + 1 < nj)
    def _prefetch():
        issue(j + 1, (j + 1) % 2)

    slot = lax.rem(j, 2)
    base = pl.multiple_of(slot * CS, 8)
    pltpu.make_async_copy(emb_hbm.at[pl.ds(0, CS), :],
                          xbuf.at[pl.ds(0, CS), :], sems.at[slot]).wait()

    x = xbuf[pl.ds(base, CS), :]

    we = jnp.maximum(
        jnp.dot(x, sW_ref[...], preferred_element_type=jnp.float32)
        + sb_ref[...], 0.0)

    top = jnp.dot(we, sdW_ref[0:256, :], preferred_element_type=jnp.float32)

    first = b0 + j * C
    n_int = sdW_ref.shape[0] - 256
    iota = lax.broadcasted_iota(jnp.int32, (1, n_int), 1)
    oh = jnp.concatenate(
        [(iota == intent_ref[first + r]).astype(jnp.float32)
         for r in range(C)], axis=0)
    bots = jnp.dot(oh, sdW_ref[256:, :], preferred_element_type=jnp.float32)

    n_slots = top.shape[1]
    top3 = top.reshape(C, S, n_slots)
    slots_out[...] = (top3 + bots[:, None, :] + sdb_ref[...]).reshape(CS, n_slots)

    m = jnp.mean(x.reshape(C, S, x.shape[1]), axis=1)
    se = jnp.maximum(
        jnp.dot(m, dW_ref[...], preferred_element_type=jnp.float32)
        + db_ref[...], 0.0)
    logits = (jnp.dot(se, ddW_ref[...], preferred_element_type=jnp.float32)
              + ddb_ref[...])
    intent_out[...] = logits.reshape(intent_out.shape)


def kernel(token_ids, all_intents, emb_table, doc_enc_W, doc_enc_b,
           slot_enc_W, slot_enc_b, doc_dec_W, doc_dec_b,
           slot_dec_W, slot_dec_b):
    B, S = token_ids.shape
    VOCAB, EMB = emb_table.shape
    ENC = doc_enc_W.shape[1]
    N_INTENTS = doc_dec_W.shape[1]
    N_SLOTS = slot_dec_W.shape[1]
    C = _CHUNK

    tok = token_ids.astype(jnp.int32)
    intents = all_intents.astype(jnp.int32)

    half = B // 2
    nj = half // C
    grid = (2, nj)

    def _fixed(c, j, *_):
        return (0, 0)

    def _slots_map(c, j, *_):
        return (c * nj + j, 0)

    def _intent_map(c, j, *_):
        return (c * nj + j, 0, 0)

    grid_spec = pltpu.PrefetchScalarGridSpec(
        num_scalar_prefetch=2,
        grid=grid,
        in_specs=[
            pl.BlockSpec(memory_space=pl.ANY),
            pl.BlockSpec((EMB, ENC), _fixed),
            pl.BlockSpec((1, ENC), _fixed),
            pl.BlockSpec((EMB, ENC), _fixed),
            pl.BlockSpec((1, ENC), _fixed),
            pl.BlockSpec((ENC, N_INTENTS), _fixed),
            pl.BlockSpec((1, N_INTENTS), _fixed),
            pl.BlockSpec((slot_dec_W.shape[0], N_SLOTS), _fixed),
            pl.BlockSpec((1, N_SLOTS), _fixed),
        ],
        out_specs=[
            pl.BlockSpec((C, 1, N_INTENTS), _intent_map),
            pl.BlockSpec((C * S, N_SLOTS), _slots_map),
        ],
        scratch_shapes=[
            pltpu.VMEM((2 * C * S, EMB), jnp.float32),
            pltpu.SemaphoreType.DMA((2,)),
        ],
    )

    intent_batch, slots_batch = pl.pallas_call(
        _body,
        out_shape=[
            jax.ShapeDtypeStruct((B, 1, N_INTENTS), jnp.float32),
            jax.ShapeDtypeStruct((B * S, N_SLOTS), jnp.float32),
        ],
        grid_spec=grid_spec,
        compiler_params=pltpu.CompilerParams(
            dimension_semantics=("parallel", "arbitrary"),
            disable_bounds_checks=True,
        ),
        name="intent_slot_fused",
    )(tok, intents, emb_table, doc_enc_W, doc_enc_b.reshape(1, ENC),
      slot_enc_W, slot_enc_b.reshape(1, ENC), doc_dec_W,
      doc_dec_b.reshape(1, N_INTENTS), slot_dec_W,
      slot_dec_b.reshape(1, N_SLOTS))
    return intent_batch.reshape(B, N_INTENTS), slots_batch


# speedup vs baseline: 5.1592x; 1.1154x over previous
import jax
import jax.numpy as jnp
from jax import lax
from jax.experimental import pallas as pl
from jax.experimental.pallas import tpu as pltpu

_CHUNK = 16


def _body(tok_ref, intent_ref, emb_hbm,
          dW_ref, db_ref, sW_ref, sb_ref, ddW_ref, ddb_ref, sdW_ref, sdb_ref,
          intent_out, slots_out, xbuf, sems):
    nj = pl.num_programs(1)
    C = _CHUNK
    S = tok_ref.shape[1]
    CS = C * S
    c = pl.program_id(0)
    j = pl.program_id(1)
    half = nj * C
    b0 = c * half

    def issue(chunk_idx, slot):
        sbase = pl.multiple_of(slot * CS, 8)
        first = b0 + chunk_idx * C
        for wi in range(CS):
            tok = tok_ref[first + wi // S, wi % S]
            pltpu.make_async_copy(
                emb_hbm.at[pl.ds(tok, 1), :],
                xbuf.at[pl.ds(sbase + wi, 1), :],
                sems.at[slot]).start()

    @pl.when(j == 0)
    def _prologue():
        issue(j, 0)

    @pl.when(j + 1 < nj)
    def _prefetch():
        issue(j + 1, (j + 1) % 2)

    slot = lax.rem(j, 2)
    base = pl.multiple_of(slot * CS, 8)
    pltpu.make_async_copy(emb_hbm.at[pl.ds(0, CS), :],
                          xbuf.at[pl.ds(0, CS), :], sems.at[slot]).wait()

    x = xbuf[pl.ds(base, CS), :]

    we = jnp.maximum(
        jnp.dot(x, sW_ref[...], preferred_element_type=jnp.float32)
        + sb_ref[...], 0.0)

    top = jnp.dot(we, sdW_ref[0:256, :], preferred_element_type=jnp.float32)

    first = b0 + j * C
    n_int = sdW_ref.shape[0] - 256
    iota = lax.broadcasted_iota(jnp.int32, (1, n_int), 1)
    oh = jnp.concatenate(
        [(iota == intent_ref[first + r]).astype(jnp.float32)
         for r in range(C)], axis=0)
    bots = jnp.dot(oh, sdW_ref[256:, :], preferred_element_type=jnp.float32)

    n_slots = top.shape[1]
    top3 = top.reshape(C, S, n_slots)
    slots_out[...] = (top3 + bots[:, None, :] + sdb_ref[...]).reshape(CS, n_slots)

    m = jnp.mean(x.reshape(C, S, x.shape[1]), axis=1)
    se = jnp.maximum(
        jnp.dot(m, dW_ref[...], preferred_element_type=jnp.float32)
        + db_ref[...], 0.0)
    logits = (jnp.dot(se, ddW_ref[...], preferred_element_type=jnp.float32)
              + ddb_ref[...])
    intent_out[...] = logits.reshape(intent_out.shape)


def kernel(token_ids, all_intents, emb_table, doc_enc_W, doc_enc_b,
           slot_enc_W, slot_enc_b, doc_dec_W, doc_dec_b,
           slot_dec_W, slot_dec_b):
    B, S = token_ids.shape
    VOCAB, EMB = emb_table.shape
    ENC = doc_enc_W.shape[1]
    N_INTENTS = doc_dec_W.shape[1]
    N_SLOTS = slot_dec_W.shape[1]
    C = _CHUNK

    tok = token_ids.astype(jnp.int32)
    intents = all_intents.astype(jnp.int32)

    half = B // 2
    nj = half // C
    grid = (2, nj)

    def _fixed(c, j, *_):
        return (0, 0)

    def _slots_map(c, j, *_):
        return (c * nj + j, 0)

    def _intent_map(c, j, *_):
        return (c * nj + j, 0, 0)

    grid_spec = pltpu.PrefetchScalarGridSpec(
        num_scalar_prefetch=2,
        grid=grid,
        in_specs=[
            pl.BlockSpec(memory_space=pl.ANY),
            pl.BlockSpec((EMB, ENC), _fixed),
            pl.BlockSpec((1, ENC), _fixed),
            pl.BlockSpec((EMB, ENC), _fixed),
            pl.BlockSpec((1, ENC), _fixed),
            pl.BlockSpec((ENC, N_INTENTS), _fixed),
            pl.BlockSpec((1, N_INTENTS), _fixed),
            pl.BlockSpec((slot_dec_W.shape[0], N_SLOTS), _fixed),
            pl.BlockSpec((1, N_SLOTS), _fixed),
        ],
        out_specs=[
            pl.BlockSpec((C, 1, N_INTENTS), _intent_map),
            pl.BlockSpec((C * S, N_SLOTS), _slots_map),
        ],
        scratch_shapes=[
            pltpu.VMEM((2 * C * S, EMB), jnp.float32),
            pltpu.SemaphoreType.DMA((2,)),
        ],
    )

    intent_batch, slots_batch = pl.pallas_call(
        _body,
        out_shape=[
            jax.ShapeDtypeStruct((B, 1, N_INTENTS), jnp.float32),
            jax.ShapeDtypeStruct((B * S, N_SLOTS), jnp.float32),
        ],
        grid_spec=grid_spec,
        compiler_params=pltpu.CompilerParams(
            dimension_semantics=("parallel", "arbitrary"),
            disable_bounds_checks=True,
        ),
        name="intent_slot_fused",
    )(tok, intents, emb_table, doc_enc_W, doc_enc_b.reshape(1, ENC),
      slot_enc_W, slot_enc_b.reshape(1, ENC), doc_dec_W,
      doc_dec_b.reshape(1, N_INTENTS), slot_dec_W,
      slot_dec_b.reshape(1, N_SLOTS))
    return intent_batch.reshape(B, N_INTENTS), slots_batch


# speedup vs baseline: 5.3087x; 1.0290x over previous
import jax
import jax.numpy as jnp
from jax import lax
from jax.experimental import pallas as pl
from jax.experimental.pallas import tpu as pltpu

_CHUNK = 32


def _body(tok_ref, intent_ref, emb_hbm,
          dW_ref, db_ref, sW_ref, sb_ref, ddW_ref, ddb_ref, sdW_ref, sdb_ref,
          intent_out, slots_out, xbuf, sems):
    nj = pl.num_programs(1)
    C = _CHUNK
    S = tok_ref.shape[1]
    CS = C * S
    c = pl.program_id(0)
    j = pl.program_id(1)
    half = nj * C
    b0 = c * half

    def issue(chunk_idx, slot):
        sbase = pl.multiple_of(slot * CS, 8)
        first = b0 + chunk_idx * C
        for wi in range(CS):
            tok = tok_ref[first + wi // S, wi % S]
            pltpu.make_async_copy(
                emb_hbm.at[pl.ds(tok, 1), :],
                xbuf.at[pl.ds(sbase + wi, 1), :],
                sems.at[slot]).start()

    @pl.when(j == 0)
    def _prologue():
        issue(j, 0)

    @pl.when(j + 1 < nj)
    def _prefetch():
        issue(j + 1, (j + 1) % 2)

    slot = lax.rem(j, 2)
    base = pl.multiple_of(slot * CS, 8)
    pltpu.make_async_copy(emb_hbm.at[pl.ds(0, CS), :],
                          xbuf.at[pl.ds(0, CS), :], sems.at[slot]).wait()

    x = xbuf[pl.ds(base, CS), :]

    we = jnp.maximum(
        jnp.dot(x, sW_ref[...], preferred_element_type=jnp.float32)
        + sb_ref[...], 0.0)

    top = jnp.dot(we, sdW_ref[0:256, :], preferred_element_type=jnp.float32)

    first = b0 + j * C
    n_int = sdW_ref.shape[0] - 256
    iota = lax.broadcasted_iota(jnp.int32, (1, n_int), 1)
    oh = jnp.concatenate(
        [(iota == intent_ref[first + r]).astype(jnp.float32)
         for r in range(C)], axis=0)
    bots = jnp.dot(oh, sdW_ref[256:, :], preferred_element_type=jnp.float32)

    n_slots = top.shape[1]
    top3 = top.reshape(C, S, n_slots)
    slots_out[...] = (top3 + bots[:, None, :] + sdb_ref[...]).reshape(CS, n_slots)

    m = jnp.mean(x.reshape(C, S, x.shape[1]), axis=1)
    se = jnp.maximum(
        jnp.dot(m, dW_ref[...], preferred_element_type=jnp.float32)
        + db_ref[...], 0.0)
    logits = (jnp.dot(se, ddW_ref[...], preferred_element_type=jnp.float32)
              + ddb_ref[...])
    intent_out[...] = logits.reshape(intent_out.shape)


def kernel(token_ids, all_intents, emb_table, doc_enc_W, doc_enc_b,
           slot_enc_W, slot_enc_b, doc_dec_W, doc_dec_b,
           slot_dec_W, slot_dec_b):
    B, S = token_ids.shape
    VOCAB, EMB = emb_table.shape
    ENC = doc_enc_W.shape[1]
    N_INTENTS = doc_dec_W.shape[1]
    N_SLOTS = slot_dec_W.shape[1]
    C = _CHUNK

    tok = token_ids.astype(jnp.int32)
    intents = all_intents.astype(jnp.int32)

    half = B // 2
    nj = half // C
    grid = (2, nj)

    def _fixed(c, j, *_):
        return (0, 0)

    def _slots_map(c, j, *_):
        return (c * nj + j, 0)

    def _intent_map(c, j, *_):
        return (c * nj + j, 0, 0)

    grid_spec = pltpu.PrefetchScalarGridSpec(
        num_scalar_prefetch=2,
        grid=grid,
        in_specs=[
            pl.BlockSpec(memory_space=pl.ANY),
            pl.BlockSpec((EMB, ENC), _fixed),
            pl.BlockSpec((1, ENC), _fixed),
            pl.BlockSpec((EMB, ENC), _fixed),
            pl.BlockSpec((1, ENC), _fixed),
            pl.BlockSpec((ENC, N_INTENTS), _fixed),
            pl.BlockSpec((1, N_INTENTS), _fixed),
            pl.BlockSpec((slot_dec_W.shape[0], N_SLOTS), _fixed),
            pl.BlockSpec((1, N_SLOTS), _fixed),
        ],
        out_specs=[
            pl.BlockSpec((C, 1, N_INTENTS), _intent_map),
            pl.BlockSpec((C * S, N_SLOTS), _slots_map),
        ],
        scratch_shapes=[
            pltpu.VMEM((2 * C * S, EMB), jnp.float32),
            pltpu.SemaphoreType.DMA((2,)),
        ],
    )

    intent_batch, slots_batch = pl.pallas_call(
        _body,
        out_shape=[
            jax.ShapeDtypeStruct((B, 1, N_INTENTS), jnp.float32),
            jax.ShapeDtypeStruct((B * S, N_SLOTS), jnp.float32),
        ],
        grid_spec=grid_spec,
        compiler_params=pltpu.CompilerParams(
            dimension_semantics=("parallel", "arbitrary"),
            disable_bounds_checks=True,
        ),
        name="intent_slot_fused",
    )(tok, intents, emb_table, doc_enc_W, doc_enc_b.reshape(1, ENC),
      slot_enc_W, slot_enc_b.reshape(1, ENC), doc_dec_W,
      doc_dec_b.reshape(1, N_INTENTS), slot_dec_W,
      slot_dec_b.reshape(1, N_SLOTS))
    return intent_batch.reshape(B, N_INTENTS), slots_batch


# speedup vs baseline: 5.6275x; 1.0600x over previous
import jax
import jax.numpy as jnp
from jax import lax
from jax.experimental import pallas as pl
from jax.experimental.pallas import tpu as pltpu

_NCH = 4


def _body(tok_ref, intent_ref, emb_hbm,
          dW_ref, db_ref, sW_ref, sb_ref, ddW_ref, ddb_ref, sdW_ref, sdb_ref,
          intent_out, slots_out, xbuf, sems):
    nj = pl.num_programs(1)
    S = tok_ref.shape[1]
    rows = xbuf.shape[0]
    CS = rows // nj
    C = CS // S
    c = pl.program_id(0)
    j = pl.program_id(1)
    b0 = c * (nj * C)

    @pl.when(j == 0)
    def _issue_all():
        for wi in range(rows):
            tok = tok_ref[b0 + wi // S, wi % S]
            pltpu.make_async_copy(
                emb_hbm.at[pl.ds(tok, 1), :],
                xbuf.at[pl.ds(wi, 1), :],
                sems.at[wi // CS]).start()

    pltpu.make_async_copy(emb_hbm.at[pl.ds(0, CS), :],
                          xbuf.at[pl.ds(0, CS), :], sems.at[j]).wait()

    base = pl.multiple_of(j * CS, 8)
    x = xbuf[pl.ds(base, CS), :]

    we = jnp.maximum(
        jnp.dot(x, sW_ref[...], preferred_element_type=jnp.float32)
        + sb_ref[...], 0.0)

    top = jnp.dot(we, sdW_ref[0:256, :], preferred_element_type=jnp.float32)

    first = b0 + j * C
    n_int = sdW_ref.shape[0] - 256
    iota = lax.broadcasted_iota(jnp.int32, (1, n_int), 1)
    oh = jnp.concatenate(
        [(iota == intent_ref[first + r]).astype(jnp.float32)
         for r in range(C)], axis=0)
    bots = jnp.dot(oh, sdW_ref[256:, :], preferred_element_type=jnp.float32)

    n_slots = top.shape[1]
    top3 = top.reshape(C, S, n_slots)
    slots_out[...] = (top3 + bots[:, None, :] + sdb_ref[...]).reshape(CS, n_slots)

    m = jnp.mean(x.reshape(C, S, x.shape[1]), axis=1)
    se = jnp.maximum(
        jnp.dot(m, dW_ref[...], preferred_element_type=jnp.float32)
        + db_ref[...], 0.0)
    logits = (jnp.dot(se, ddW_ref[...], preferred_element_type=jnp.float32)
              + ddb_ref[...])
    intent_out[...] = logits.reshape(intent_out.shape)


def kernel(token_ids, all_intents, emb_table, doc_enc_W, doc_enc_b,
           slot_enc_W, slot_enc_b, doc_dec_W, doc_dec_b,
           slot_dec_W, slot_dec_b):
    B, S = token_ids.shape
    VOCAB, EMB = emb_table.shape
    ENC = doc_enc_W.shape[1]
    N_INTENTS = doc_dec_W.shape[1]
    N_SLOTS = slot_dec_W.shape[1]
    nj = _NCH
    half = B // 2
    C = half // nj

    tok = token_ids.astype(jnp.int32)
    intents = all_intents.astype(jnp.int32)

    grid = (2, nj)

    def _fixed(c, j, *_):
        return (0, 0)

    def _slots_map(c, j, *_):
        return (c * nj + j, 0)

    def _intent_map(c, j, *_):
        return (c * nj + j, 0, 0)

    grid_spec = pltpu.PrefetchScalarGridSpec(
        num_scalar_prefetch=2,
        grid=grid,
        in_specs=[
            pl.BlockSpec(memory_space=pl.ANY),
            pl.BlockSpec((EMB, ENC), _fixed),
            pl.BlockSpec((1, ENC), _fixed),
            pl.BlockSpec((EMB, ENC), _fixed),
            pl.BlockSpec((1, ENC), _fixed),
            pl.BlockSpec((ENC, N_INTENTS), _fixed),
            pl.BlockSpec((1, N_INTENTS), _fixed),
            pl.BlockSpec((slot_dec_W.shape[0], N_SLOTS), _fixed),
            pl.BlockSpec((1, N_SLOTS), _fixed),
        ],
        out_specs=[
            pl.BlockSpec((C, 1, N_INTENTS), _intent_map),
            pl.BlockSpec((C * S, N_SLOTS), _slots_map),
        ],
        scratch_shapes=[
            pltpu.VMEM((half * S, EMB), jnp.float32),
            pltpu.SemaphoreType.DMA((nj,)),
        ],
    )

    intent_batch, slots_batch = pl.pallas_call(
        _body,
        out_shape=[
            jax.ShapeDtypeStruct((B, 1, N_INTENTS), jnp.float32),
            jax.ShapeDtypeStruct((B * S, N_SLOTS), jnp.float32),
        ],
        grid_spec=grid_spec,
        compiler_params=pltpu.CompilerParams(
            dimension_semantics=("parallel", "arbitrary"),
            disable_bounds_checks=True,
        ),
        name="intent_slot_fused",
    )(tok, intents, emb_table, doc_enc_W, doc_enc_b.reshape(1, ENC),
      slot_enc_W, slot_enc_b.reshape(1, ENC), doc_dec_W,
      doc_dec_b.reshape(1, N_INTENTS), slot_dec_W,
      slot_dec_b.reshape(1, N_SLOTS))
    return intent_batch.reshape(B, N_INTENTS), slots_batch
